# baseline (speedup 1.0000x reference)
import jax
import jax.numpy as jnp
from jax import lax
from jax.experimental import pallas as pl
from jax.experimental.pallas import tpu as pltpu
from jax.experimental.pallas import tpu_sc as plsc

N = 10
M = 64
KPTS = 16
LANES = 16
GROUPS = M // LANES
TASKS = N * GROUPS
NWORKERS = 32
XWORDS = N * 3 * LANES
GWORDS = KPTS * 3 * LANES


def _sc_body(xb_hbm, kt_hbm, out_hbm, xv, kv, ov, sem0, sem1):
    w = lax.axis_index("s") * 2 + lax.axis_index("c")
    second = w < TASKS - NWORKERS

    def fire(t, slot, sem):
        i = t // GROUPS
        g = t - i * GROUPS
        cx = pltpu.make_async_copy(xb_hbm.at[pl.ds(i * 48, 48)],
                                   xv.at[pl.ds(slot * 48, 48)], sem)
        ck = pltpu.make_async_copy(kt_hbm.at[pl.ds(g * GWORDS, GWORDS)],
                                   kv.at[pl.ds(slot * GWORDS, GWORDS)], sem)
        cx.start()
        ck.start()
        return cx, ck

    def compute(t, slot, copies):
        i = t // GROUPS
        g = t - i * GROUPS
        cx, ck = copies
        cx.wait()
        ck.wait()
        x0 = xv[pl.ds(slot * 48 + 0 * LANES, LANES)]
        x1 = xv[pl.ds(slot * 48 + 1 * LANES, LANES)]
        x2 = xv[pl.ds(slot * 48 + 2 * LANES, LANES)]
        acc = None
        for l in range(KPTS):
            d0 = x0 - kv[pl.ds(slot * GWORDS + (l * 3 + 0) * LANES, LANES)]
            d1 = x1 - kv[pl.ds(slot * GWORDS + (l * 3 + 1) * LANES, LANES)]
            d2 = x2 - kv[pl.ds(slot * GWORDS + (l * 3 + 2) * LANES, LANES)]
            e = jnp.exp(-(d0 * d0 + d1 * d1 + d2 * d2))
            acc = e if acc is None else acc + e
        ov[...] = acc * (1.0 / 128.0)
        pltpu.sync_copy(ov, out_hbm.at[i, pl.ds(g * LANES, LANES)])

    c0 = fire(w, 0, sem0)

    @pl.when(second)
    def _fire2():
        fire(w + NWORKERS, 1, sem1)

    compute(w, 0, c0)

    @pl.when(second)
    def _compute2():
        t2 = w + NWORKERS
        i2 = t2 // GROUPS
        g2 = t2 - i2 * GROUPS
        cx2 = pltpu.make_async_copy(xb_hbm.at[pl.ds(i2 * 48, 48)],
                                    xv.at[pl.ds(48, 48)], sem1)
        ck2 = pltpu.make_async_copy(kt_hbm.at[pl.ds(g2 * GWORDS, GWORDS)],
                                    kv.at[pl.ds(GWORDS, GWORDS)], sem1)
        compute(t2, 1, (cx2, ck2))


@jax.jit
def _run(normal, learnable_kernel):
    xb = jnp.broadcast_to(normal[:N, :, None], (N, 3, LANES)).reshape(XWORDS)
    kt = (learnable_kernel.reshape(GROUPS, LANES, KPTS, 3)
          .transpose(0, 2, 3, 1)
          .reshape(GROUPS * GWORDS))
    sc_call = pl.kernel(
        _sc_body,
        out_type=jax.ShapeDtypeStruct((N, M), jnp.float32),
        mesh=plsc.VectorSubcoreMesh(core_axis_name="c", subcore_axis_name="s"),
        scratch_types=[
            pltpu.VMEM((2 * 48,), jnp.float32),
            pltpu.VMEM((2 * GWORDS,), jnp.float32),
            pltpu.VMEM((LANES,), jnp.float32),
            pltpu.SemaphoreType.DMA,
            pltpu.SemaphoreType.DMA,
        ],
    )
    return sc_call(xb, kt)


def kernel(normal, neighbour, learnable_kernel):
    del neighbour
    return _run(normal, learnable_kernel)


# speedup vs baseline: 1.0215x; 1.0215x over previous
import jax
import jax.numpy as jnp
from jax import lax
from jax.experimental import pallas as pl
from jax.experimental.pallas import tpu as pltpu
from jax.experimental.pallas import tpu_sc as plsc

N = 10
M = 64
KPTS = 16
LANES = 16
GROUPS = M // LANES
TASKS = N * GROUPS
NWORKERS = 32
GWORDS = KPTS * 3 * LANES
KTOT = GROUPS * GWORDS
XOFF = KTOT
XCOPY = 32


def _sc_body(packed_hbm, out_hbm, xv, kv, ov, sem0, sem1):
    w = lax.axis_index("s") * 2 + lax.axis_index("c")
    second = w < TASKS - NWORKERS

    def kt_copy(t, slot, sem):
        g = t % GROUPS
        return pltpu.make_async_copy(packed_hbm.at[pl.ds(g * GWORDS, GWORDS)],
                                     kv.at[pl.ds(slot * GWORDS, GWORDS)], sem)

    def compute(t, slot):
        i = t // GROUPS
        g = t - i * GROUPS
        xr = xv[pl.ds(i * 3, LANES)]
        x0 = jnp.full((LANES,), xr[0], jnp.float32)
        x1 = jnp.full((LANES,), xr[1], jnp.float32)
        x2 = jnp.full((LANES,), xr[2], jnp.float32)
        acc = None
        for l in range(KPTS):
            d0 = x0 - kv[pl.ds(slot * GWORDS + (l * 3 + 0) * LANES, LANES)]
            d1 = x1 - kv[pl.ds(slot * GWORDS + (l * 3 + 1) * LANES, LANES)]
            d2 = x2 - kv[pl.ds(slot * GWORDS + (l * 3 + 2) * LANES, LANES)]
            e = jnp.exp(-(d0 * d0 + d1 * d1 + d2 * d2))
            acc = e if acc is None else acc + e
        ov[...] = acc * (1.0 / 128.0)
        pltpu.sync_copy(ov, out_hbm.at[i, pl.ds(g * LANES, LANES)])

    cx = pltpu.make_async_copy(packed_hbm.at[pl.ds(XOFF, XCOPY)],
                               xv.at[pl.ds(0, XCOPY)], sem0)
    cx.start()
    ck0 = kt_copy(w, 0, sem0)
    ck0.start()

    @pl.when(second)
    def _fire2():
        kt_copy(w + NWORKERS, 1, sem1).start()

    cx.wait()
    ck0.wait()
    compute(w, 0)

    @pl.when(second)
    def _compute2():
        kt_copy(w + NWORKERS, 1, sem1).wait()
        compute(w + NWORKERS, 1)


@jax.jit
def _run(normal, learnable_kernel):
    kt = (learnable_kernel.reshape(GROUPS, LANES, KPTS, 3)
          .transpose(0, 2, 3, 1)
          .reshape(KTOT))
    packed = jnp.concatenate([kt, normal[:N].reshape(N * 3),
                              jnp.zeros(2, jnp.float32)])
    sc_call = pl.kernel(
        _sc_body,
        out_type=jax.ShapeDtypeStruct((N, M), jnp.float32),
        mesh=plsc.VectorSubcoreMesh(core_axis_name="c", subcore_axis_name="s"),
        scratch_types=[
            pltpu.VMEM((48,), jnp.float32),
            pltpu.VMEM((2 * GWORDS,), jnp.float32),
            pltpu.VMEM((LANES,), jnp.float32),
            pltpu.SemaphoreType.DMA,
            pltpu.SemaphoreType.DMA,
        ],
    )
    return sc_call(packed)


def kernel(normal, neighbour, learnable_kernel):
    del neighbour
    return _run(normal, learnable_kernel)


# speedup vs baseline: 1.0857x; 1.0628x over previous
import jax
import jax.numpy as jnp
from jax import lax
from jax.experimental import pallas as pl
from jax.experimental.pallas import tpu as pltpu
from jax.experimental.pallas import tpu_sc as plsc

N = 10
M = 64
KPTS = 16
LANES = 16
GROUPS = M // LANES
TASKS = N * GROUPS
NW = 16
GWORDS = KPTS * 3 * LANES
KTOT = GROUPS * GWORDS
XOFF = KTOT
XCOPY = 32


def _sc_body(packed_hbm, out_hbm, xv, kv, ov, sem0, sem1, sem2):
    w = lax.axis_index("s")
    sems = (sem0, sem1, sem2)

    def kt_copy(t, slot, sem):
        g = t % GROUPS
        return pltpu.make_async_copy(packed_hbm.at[pl.ds(g * GWORDS, GWORDS)],
                                     kv.at[pl.ds(slot * GWORDS, GWORDS)], sem)

    def compute(t, slot):
        i = t // GROUPS
        g = t - i * GROUPS
        xr = xv[pl.ds(i * 3, LANES)]
        x0 = jnp.full((LANES,), xr[0], jnp.float32)
        x1 = jnp.full((LANES,), xr[1], jnp.float32)
        x2 = jnp.full((LANES,), xr[2], jnp.float32)
        acc = None
        for l in range(KPTS):
            d0 = x0 - kv[pl.ds(slot * GWORDS + (l * 3 + 0) * LANES, LANES)]
            d1 = x1 - kv[pl.ds(slot * GWORDS + (l * 3 + 1) * LANES, LANES)]
            d2 = x2 - kv[pl.ds(slot * GWORDS + (l * 3 + 2) * LANES, LANES)]
            e = jnp.exp(-(d0 * d0 + d1 * d1 + d2 * d2))
            acc = e if acc is None else acc + e
        ov[...] = acc * (1.0 / 128.0)
        pltpu.sync_copy(ov, out_hbm.at[i, pl.ds(g * LANES, LANES)])

    cx = pltpu.make_async_copy(packed_hbm.at[pl.ds(XOFF, XCOPY)],
                               xv.at[pl.ds(0, XCOPY)], sem0)
    cx.start()
    kt_copy(w, 0, sem0).start()
    kt_copy(w + NW, 1, sem1).start()

    @pl.when(w < TASKS - 2 * NW)
    def _fire3():
        kt_copy(w + 2 * NW, 2, sem2).start()

    cx.wait()
    kt_copy(w, 0, sem0).wait()
    compute(w, 0)
    kt_copy(w + NW, 1, sem1).wait()
    compute(w + NW, 1)

    @pl.when(w < TASKS - 2 * NW)
    def _compute3():
        kt_copy(w + 2 * NW, 2, sem2).wait()
        compute(w + 2 * NW, 2)


@jax.jit
def _run(normal, learnable_kernel):
    kt = (learnable_kernel.reshape(GROUPS, LANES, KPTS, 3)
          .transpose(0, 2, 3, 1)
          .reshape(KTOT))
    packed = jnp.concatenate([kt, normal[:N].reshape(N * 3),
                              jnp.zeros(2, jnp.float32)])
    sc_call = pl.kernel(
        _sc_body,
        out_type=jax.ShapeDtypeStruct((N, M), jnp.float32),
        mesh=plsc.VectorSubcoreMesh(core_axis_name="c", subcore_axis_name="s",
                                    num_cores=1),
        scratch_types=[
            pltpu.VMEM((48,), jnp.float32),
            pltpu.VMEM((3 * GWORDS,), jnp.float32),
            pltpu.VMEM((LANES,), jnp.float32),
            pltpu.SemaphoreType.DMA,
            pltpu.SemaphoreType.DMA,
            pltpu.SemaphoreType.DMA,
        ],
    )
    return sc_call(packed)


def kernel(normal, neighbour, learnable_kernel):
    del neighbour
    return _run(normal, learnable_kernel)
